# baseline (speedup 1.0000x reference)
import functools

import jax
import jax.numpy as jnp
from jax import lax
from jax.experimental import pallas as pl
from jax.experimental.pallas import tpu as pltpu
from jax.experimental.pallas import tpu_sc as plsc

BATCH = 16384
EMBED = 32
NUM_CORES = 2
NUM_SUBCORES = 16
NUM_WORKERS = NUM_CORES * NUM_SUBCORES
RPW = BATCH // NUM_WORKERS
CHUNK = 128
NUM_CHUNKS = RPW // CHUNK
LANES = 16
GROUPS = RPW // LANES


def _funk_body(uid_hbm, iid_hbm, p_hbm, q_hbm, out_hbm,
               uidx, iidx, pu, qi, yv, sem_p, sem_q):
  wid = lax.axis_index("s") * NUM_CORES + lax.axis_index("c")
  base = wid * RPW

  pltpu.sync_copy(uid_hbm.at[pl.ds(base, RPW)], uidx)
  pltpu.sync_copy(iid_hbm.at[pl.ds(base, RPW)], iidx)

  copies = []
  for j in range(NUM_CHUNKS):
    sl = pl.ds(j * CHUNK, CHUNK)
    copies.append(pltpu.async_copy(p_hbm.at[uidx.at[sl]], pu.at[sl], sem_p))
    copies.append(pltpu.async_copy(q_hbm.at[iidx.at[sl]], qi.at[sl], sem_q))
  for c in copies:
    c.wait()

  lane = lax.iota(jnp.int32, LANES)
  masks = [lane == k for k in range(LANES)]

  def group(g, carry):
    r0 = g * LANES
    acc = jnp.zeros((LANES,), jnp.float32)
    for k in range(LANES):
      r = r0 + k
      pe, po = plsc.unpack(pu[r, pl.ds(0, EMBED)],
                           format=plsc.PackFormat.INTERLEAVED)
      qe, qo = plsc.unpack(qi[r, pl.ds(0, EMBED)],
                           format=plsc.PackFormat.INTERLEAVED)
      m = pe * qe + po * qo
      s = jnp.sum(m, axis=0)
      acc = jnp.where(masks[k], jnp.broadcast_to(s, (LANES,)), acc)
    yv[pl.ds(r0, LANES)] = acc
    return carry

  lax.fori_loop(0, GROUPS, group, 0)

  pltpu.sync_copy(yv, out_hbm.at[pl.ds(base, RPW)])


_funk = functools.partial(
    pl.kernel,
    out_type=jax.ShapeDtypeStruct((BATCH,), jnp.float32),
    mesh=plsc.VectorSubcoreMesh(core_axis_name="c", subcore_axis_name="s"),
    compiler_params=pltpu.CompilerParams(
        needs_layout_passes=False, use_tc_tiling_on_sc=False),
    scratch_types=[
        pltpu.VMEM((RPW,), jnp.int32),
        pltpu.VMEM((RPW,), jnp.int32),
        pltpu.VMEM((RPW, EMBED), jnp.bfloat16),
        pltpu.VMEM((RPW, EMBED), jnp.bfloat16),
        pltpu.VMEM((RPW,), jnp.float32),
        pltpu.SemaphoreType.DMA,
        pltpu.SemaphoreType.DMA,
    ],
)(_funk_body)


@jax.jit
def kernel(user_ids, item_ids, P, Q):
  return _funk(user_ids.astype(jnp.int32), item_ids.astype(jnp.int32),
               P.astype(jnp.bfloat16), Q.astype(jnp.bfloat16))


# speedup vs baseline: 1.1722x; 1.1722x over previous
import functools

import jax
import jax.numpy as jnp
from jax import lax
from jax.experimental import pallas as pl
from jax.experimental.pallas import tpu as pltpu
from jax.experimental.pallas import tpu_sc as plsc

M_USERS = 1000000
N_ITEMS = 100000
BATCH = 16384
EMBED = 32
ROWS_PER_GROUP = 128 // EMBED
NUM_CORES = 2
NUM_SUBCORES = 16
NUM_WORKERS = NUM_CORES * NUM_SUBCORES
RPW = BATCH // NUM_WORKERS
LANES = 16
PASS = 256
NUM_PASSES = RPW // PASS
CHUNK = 128
PASS_CHUNKS = PASS // CHUNK
PASS_GROUPS = PASS // LANES


def _funk_body(uid_hbm, iid_hbm, p_hbm, q_hbm, out_hbm,
               uidx, iidx, urow, irow, pu, qi, yv, sem_p, sem_q):
  wid = lax.axis_index("s") * NUM_CORES + lax.axis_index("c")
  base = wid * RPW

  pltpu.sync_copy(uid_hbm.at[pl.ds(base, RPW)], uidx)
  pltpu.sync_copy(iid_hbm.at[pl.ds(base, RPW)], iidx)

  def rows(g, carry):
    sl = pl.ds(g * LANES, LANES)
    urow[sl] = uidx[sl] >> 2
    irow[sl] = iidx[sl] >> 2
    return carry

  lax.fori_loop(0, RPW // LANES, rows, 0)

  lane = lax.iota(jnp.int32, LANES)
  masks = [lane == k for k in range(LANES)]

  def do_pass(p, carry):
    p0 = p * PASS
    copies = []
    for c in range(PASS_CHUNKS):
      src = pl.ds(p0 + c * CHUNK, CHUNK)
      dst = pl.ds(c * CHUNK, CHUNK)
      copies.append(
          pltpu.async_copy(p_hbm.at[urow.at[src]], pu.at[dst], sem_p))
      copies.append(
          pltpu.async_copy(q_hbm.at[irow.at[src]], qi.at[dst], sem_q))
    for cp in copies:
      cp.wait()

    def group(g, carry2):
      r0 = g * LANES
      uv = uidx[pl.ds(p0 + r0, LANES)]
      iv = iidx[pl.ds(p0 + r0, LANES)]
      uoff = (uv & 3) << 5
      ioff = (iv & 3) << 5
      acc = jnp.zeros((LANES,), jnp.float32)
      for k in range(LANES):
        r = r0 + k
        uo = jnp.sum(jnp.where(masks[k], uoff, 0), axis=0)
        io = jnp.sum(jnp.where(masks[k], ioff, 0), axis=0)
        m = (pu[r, pl.ds(uo, LANES)] * qi[r, pl.ds(io, LANES)] +
             pu[r, pl.ds(uo + LANES, LANES)] * qi[r, pl.ds(io + LANES, LANES)])
        s = jnp.sum(m, axis=0)
        acc = jnp.where(masks[k], jnp.broadcast_to(s, (LANES,)), acc)
      yv[pl.ds(p0 + r0, LANES)] = acc
      return carry2

    lax.fori_loop(0, PASS_GROUPS, group, 0)
    return carry

  lax.fori_loop(0, NUM_PASSES, do_pass, 0)

  pltpu.sync_copy(yv, out_hbm.at[pl.ds(base, RPW)])


_funk = functools.partial(
    pl.kernel,
    out_type=jax.ShapeDtypeStruct((BATCH,), jnp.float32),
    mesh=plsc.VectorSubcoreMesh(core_axis_name="c", subcore_axis_name="s"),
    compiler_params=pltpu.CompilerParams(
        needs_layout_passes=False, use_tc_tiling_on_sc=False),
    scratch_types=[
        pltpu.VMEM((RPW,), jnp.int32),
        pltpu.VMEM((RPW,), jnp.int32),
        pltpu.VMEM((RPW,), jnp.int32),
        pltpu.VMEM((RPW,), jnp.int32),
        pltpu.VMEM((PASS, 128), jnp.float32),
        pltpu.VMEM((PASS, 128), jnp.float32),
        pltpu.VMEM((RPW,), jnp.float32),
        pltpu.SemaphoreType.DMA,
        pltpu.SemaphoreType.DMA,
    ],
)(_funk_body)


@jax.jit
def kernel(user_ids, item_ids, P, Q):
  p4 = P.reshape(M_USERS * EMBED // 128, 128)
  q4 = Q.reshape(N_ITEMS * EMBED // 128, 128)
  return _funk(user_ids.astype(jnp.int32), item_ids.astype(jnp.int32),
               p4, q4)


# speedup vs baseline: 2.8990x; 2.4732x over previous
import functools

import jax
import jax.numpy as jnp
from jax import lax
from jax.experimental import pallas as pl
from jax.experimental.pallas import tpu as pltpu
from jax.experimental.pallas import tpu_sc as plsc

M_USERS = 1000000
N_ITEMS = 100000
BATCH = 16384
EMBED = 32
NUM_CORES = 2
NUM_SUBCORES = 16
NUM_WORKERS = NUM_CORES * NUM_SUBCORES
RPW = BATCH // NUM_WORKERS
LANES = 16
WAVES = RPW // LANES
HALF = 8
QPASS = 128
NUM_QPASSES = RPW // QPASS
CHUNK = 128
QPASS_CHUNKS = QPASS // CHUNK
Q_ROWS = N_ITEMS * EMBED // 128


def _funk_body(uid_hbm, iid_hbm, pt_hbm, q4_hbm, out_hbm,
               uidx, iidx, irow, win, qd, prows, yv, sem_p, sem_q):
  wid = lax.axis_index("s") * NUM_CORES + lax.axis_index("c")
  base = wid * RPW

  pltpu.sync_copy(uid_hbm.at[pl.ds(base, RPW)], uidx)
  pltpu.sync_copy(iid_hbm.at[pl.ds(base, RPW)], iidx)

  def rows(g, carry):
    sl = pl.ds(g * LANES, LANES)
    irow[sl] = iidx[sl] >> 2
    return carry

  lax.fori_loop(0, RPW // LANES, rows, 0)

  lane = lax.iota(jnp.int32, LANES)
  masks = [lane == k for k in range(LANES)]

  def q_fire(p):
    copies = []
    for c in range(QPASS_CHUNKS):
      src = pl.ds(p * QPASS + c * CHUNK, CHUNK)
      dst = pl.ds(c * CHUNK, CHUNK)
      copies.append(
          pltpu.async_copy(q4_hbm.at[irow.at[src]], qd.at[dst], sem_q))
    return copies

  q_copies = q_fire(0)

  def wave(g, carry):
    uv = uidx[pl.ds(g * LANES, LANES)]
    for h in range(LANES // HALF):
      uks = []
      copies = []
      for s in range(HALF):
        k = h * HALF + s
        u_k = jnp.sum(jnp.where(masks[k], uv, 0), axis=0)
        uks.append(u_k)
        cb = pl.multiple_of((u_k >> 7) * 128, 128)
        copies.append(pltpu.async_copy(
            pt_hbm.at[pl.ds(0, EMBED), pl.ds(cb, 128)],
            win.at[pl.ds(0, EMBED), pl.ds(s * 128, 128)], sem_p))
      for cp in copies:
        cp.wait()
      for s in range(HALF):
        k = h * HALF + s
        col = jnp.full((LANES,), s * 128, jnp.int32) + (uks[s] & 127)
        g1 = plsc.load_gather(win, [lane, col])
        g2 = plsc.load_gather(win, [lane + LANES, col])
        r = g * LANES + k
        prows[r, pl.ds(0, LANES)] = g1
        prows[r, pl.ds(LANES, LANES)] = g2
    return carry

  lax.fori_loop(0, WAVES, wave, 0)

  def dot_pass(p, q_wait):
    for cp in q_wait:
      cp.wait()

    def group(g, carry):
      r0 = p * QPASS + g * LANES
      iv = iidx[pl.ds(r0, LANES)]
      ioff = (iv & 3) << 5
      acc = jnp.zeros((LANES,), jnp.float32)
      for k in range(LANES):
        r = r0 + k
        rq = g * LANES + k
        io = jnp.sum(jnp.where(masks[k], ioff, 0), axis=0)
        m = (prows[r, pl.ds(0, LANES)] * qd[rq, pl.ds(io, LANES)] +
             prows[r, pl.ds(LANES, LANES)] * qd[rq, pl.ds(io + LANES, LANES)])
        s = jnp.sum(m, axis=0)
        acc = jnp.where(masks[k], jnp.broadcast_to(s, (LANES,)), acc)
      yv[pl.ds(r0, LANES)] = acc
      return carry

    lax.fori_loop(0, QPASS // LANES, group, 0)

  for p in range(NUM_QPASSES):
    dot_pass(p, q_copies)
    if p + 1 < NUM_QPASSES:
      q_copies = q_fire(p + 1)

  pltpu.sync_copy(yv, out_hbm.at[pl.ds(base, RPW)])


_funk = functools.partial(
    pl.kernel,
    out_type=jax.ShapeDtypeStruct((BATCH,), jnp.float32),
    mesh=plsc.VectorSubcoreMesh(core_axis_name="c", subcore_axis_name="s"),
    compiler_params=pltpu.CompilerParams(needs_layout_passes=False),
    scratch_types=[
        pltpu.VMEM((RPW,), jnp.int32),
        pltpu.VMEM((RPW,), jnp.int32),
        pltpu.VMEM((RPW,), jnp.int32),
        pltpu.VMEM((EMBED, HALF * 128), jnp.float32),
        pltpu.VMEM((QPASS, 128), jnp.float32),
        pltpu.VMEM((RPW, EMBED), jnp.float32),
        pltpu.VMEM((RPW,), jnp.float32),
        pltpu.SemaphoreType.DMA,
        pltpu.SemaphoreType.DMA,
    ],
)(_funk_body)


@jax.jit
def kernel(user_ids, item_ids, P, Q):
  q4 = Q.reshape(Q_ROWS, 128)
  return _funk(user_ids.astype(jnp.int32), item_ids.astype(jnp.int32),
               P.T, q4)


# speedup vs baseline: 3.1525x; 1.0874x over previous
import functools

import jax
import jax.numpy as jnp
from jax import lax
from jax.experimental import pallas as pl
from jax.experimental.pallas import tpu as pltpu
from jax.experimental.pallas import tpu_sc as plsc

M_USERS = 1000000
N_ITEMS = 100000
BATCH = 16384
EMBED = 32
NUM_CORES = 2
NUM_SUBCORES = 16
NUM_WORKERS = NUM_CORES * NUM_SUBCORES
RPW = BATCH // NUM_WORKERS
LANES = 16
HALF = 8
BLOCK = 64
NUM_BLOCKS = RPW // BLOCK
BLOCK_WAVES = BLOCK // LANES
Q_ROWS = N_ITEMS * EMBED // 128


def _funk_body(uid_hbm, iid_hbm, pt_hbm, q4_hbm, out_hbm,
               uidx, iidx, irow, win, qd, prows, yv, sem_p, sem_p2, sem_q):
  wid = lax.axis_index("s") * NUM_CORES + lax.axis_index("c")
  base = wid * RPW

  pltpu.sync_copy(uid_hbm.at[pl.ds(base, RPW)], uidx)
  pltpu.sync_copy(iid_hbm.at[pl.ds(base, RPW)], iidx)

  def rows(g, carry):
    sl = pl.ds(g * LANES, LANES)
    irow[sl] = iidx[sl] >> 2
    return carry

  lax.fori_loop(0, RPW // LANES, rows, 0)

  lane = lax.iota(jnp.int32, LANES)
  masks = [lane == k for k in range(LANES)]
  sems = [sem_p, sem_p2]

  def q_fire(b):
    return pltpu.async_copy(
        q4_hbm.at[irow.at[pl.ds(b * BLOCK, BLOCK)]], qd, sem_q)

  def wave(b, gl):
    uv = uidx[pl.ds(b * BLOCK + gl * LANES, LANES)]
    uks = []
    copies = []
    for k in range(LANES):
      u_k = jnp.sum(jnp.where(masks[k], uv, 0), axis=0)
      uks.append(u_k)
      cb = pl.multiple_of((u_k >> 7) * 128, 128)
      copies.append(pltpu.async_copy(
          pt_hbm.at[pl.ds(0, EMBED), pl.ds(cb, 128)],
          win.at[pl.ds(0, EMBED), pl.ds(k * 128, 128)],
          sems[k // HALF]))
    for h in range(LANES // HALF):
      for s in range(HALF):
        copies[h * HALF + s].wait()
      for s in range(HALF):
        k = h * HALF + s
        col = jnp.full((LANES,), k * 128, jnp.int32) + (uks[k] & 127)
        g1 = plsc.load_gather(win, [lane, col])
        g2 = plsc.load_gather(win, [lane + LANES, col])
        r = gl * LANES + k
        prows[r, pl.ds(0, LANES)] = g1
        prows[r, pl.ds(LANES, LANES)] = g2

  def dot_group(b, gl):
    r0 = gl * LANES
    iv = iidx[pl.ds(b * BLOCK + r0, LANES)]
    ioff = (iv & 3) << 5
    acc = jnp.zeros((LANES,), jnp.float32)
    for k in range(LANES):
      r = r0 + k
      io = jnp.sum(jnp.where(masks[k], ioff, 0), axis=0)
      m = (prows[r, pl.ds(0, LANES)] * qd[r, pl.ds(io, LANES)] +
           prows[r, pl.ds(LANES, LANES)] * qd[r, pl.ds(io + LANES, LANES)])
      s = jnp.sum(m, axis=0)
      acc = jnp.where(masks[k], jnp.broadcast_to(s, (LANES,)), acc)
    yv[pl.ds(b * BLOCK + r0, LANES)] = acc

  def block(b, carry):
    qc = q_fire(b)
    for gl in range(BLOCK_WAVES):
      wave(b, gl)
    qc.wait()
    for gl in range(BLOCK_WAVES):
      dot_group(b, gl)
    return carry

  lax.fori_loop(0, NUM_BLOCKS, block, 0)

  pltpu.sync_copy(yv, out_hbm.at[pl.ds(base, RPW)])


_funk = functools.partial(
    pl.kernel,
    out_type=jax.ShapeDtypeStruct((BATCH,), jnp.float32),
    mesh=plsc.VectorSubcoreMesh(core_axis_name="c", subcore_axis_name="s"),
    compiler_params=pltpu.CompilerParams(needs_layout_passes=False),
    scratch_types=[
        pltpu.VMEM((RPW,), jnp.int32),
        pltpu.VMEM((RPW,), jnp.int32),
        pltpu.VMEM((RPW,), jnp.int32),
        pltpu.VMEM((EMBED, LANES * 128), jnp.float32),
        pltpu.VMEM((BLOCK, 128), jnp.float32),
        pltpu.VMEM((BLOCK, EMBED), jnp.float32),
        pltpu.VMEM((RPW,), jnp.float32),
        pltpu.SemaphoreType.DMA,
        pltpu.SemaphoreType.DMA,
        pltpu.SemaphoreType.DMA,
    ],
)(_funk_body)


@jax.jit
def kernel(user_ids, item_ids, P, Q):
  q4 = Q.reshape(Q_ROWS, 128)
  return _funk(user_ids.astype(jnp.int32), item_ids.astype(jnp.int32),
               P.T, q4)
